# baseline (speedup 1.0000x reference)
import functools

import jax
import jax.numpy as jnp
from jax import lax
from jax.experimental import pallas as pl
from jax.experimental.pallas import tpu as pltpu
from jax.experimental.pallas import tpu_sc as plsc

E_OLD = 640000
E_KEPT = 320000
D = 128

NC = 2
NS = 16
NW = NC * NS
CAND = E_OLD // NW
KPW = E_KEPT // NW
SCAT = 100
BLK = 4
CBLK = BLK * SCAT
KNB = KPW // CBLK
WIN = 20480
WINP = 20576
FLAGP = 20096
CCAP = 20800

_mesh = plsc.VectorSubcoreMesh(core_axis_name="c", subcore_axis_name="s")


def _build_body(mask_hbm, bounds_hbm, comp_hbm, counts_hbm,
                bounds_v, window_v, flag_v, comp_v, cnt_v):
    wid = lax.axis_index("s") * NC + lax.axis_index("c")
    pltpu.sync_copy(bounds_hbm, bounds_v)
    bvec = bounds_v[pl.ds(wid, 16)]
    p0 = bvec[0]
    p1 = bvec[1]
    base = wid * CAND

    b0 = pl.multiple_of(lax.min((p0 // 8) * 8, E_KEPT - WIN), 8)
    pltpu.sync_copy(mask_hbm.at[pl.ds(b0, WIN)], window_v.at[pl.ds(0, WIN)])

    zeros16 = jnp.zeros((16,), jnp.int32)
    ones16 = jnp.ones((16,), jnp.int32)
    iota16 = lax.iota(jnp.int32, 16)

    def zero_step(t, _):
        for u in range(8):
            flag_v[pl.ds(128 * t + 16 * u, 16)] = zeros16
        return 0

    lax.fori_loop(0, FLAGP // 128, zero_step, 0)

    q0 = p0 - b0
    lim = p1 - b0

    def mark_step(t, _):
        for u in range(4):
            pos = q0 + 64 * t + 16 * u + iota16
            e = plsc.load_gather(window_v, [pos])
            valid = pos < lim
            rel = jnp.where(valid, e - base, 0)
            plsc.store_scatter(flag_v, [rel], ones16, mask=valid)
        return 0

    lax.fori_loop(0, (p1 - p0 + 63) // 64, mark_step, 0)

    def comp_step(t, cptr):
        for u in range(2):
            off = 32 * t + 16 * u
            notkept = flag_v[pl.ds(off, 16)] == 0
            v = base + off + iota16
            plsc.store_compressed(comp_v.at[pl.ds(cptr, 16)], v, mask=notkept)
            cptr = cptr + plsc.all_reduce_population_count(notkept)[0]
        return cptr

    c_total = lax.fori_loop(0, CAND // 32, comp_step, 0)

    nb = (c_total + CBLK - 1) // CBLK
    nb = nb + (nb % 2)

    @pl.when(c_total > 0)
    def _pad():
        lastv = plsc.load_gather(
            comp_v, [jnp.full((16,), c_total - 1, jnp.int32)])

        def pad_step(t, _):
            comp_v[pl.ds(c_total + 16 * t, 16)] = lastv
            return 0

        lax.fori_loop(0, (nb * CBLK - c_total + 15) // 16, pad_step, 0)

    cnt_v[pl.ds(0, 16)] = jnp.full((16,), nb, jnp.int32)
    pltpu.sync_copy(comp_v, comp_hbm.at[wid])
    pltpu.sync_copy(cnt_v, counts_hbm.at[wid])


_build = functools.partial(
    pl.kernel,
    mesh=_mesh,
    compiler_params=pltpu.CompilerParams(needs_layout_passes=False),
    out_type=(
        jax.ShapeDtypeStruct((NW, CCAP), jnp.int32),
        jax.ShapeDtypeStruct((NW, 16), jnp.int32),
    ),
    scratch_types=[
        pltpu.VMEM((48,), jnp.int32),
        pltpu.VMEM((WINP,), jnp.int32),
        pltpu.VMEM((FLAGP,), jnp.int32),
        pltpu.VMEM((CCAP,), jnp.int32),
        pltpu.VMEM((16,), jnp.int32),
    ],
)(_build_body)


def _scatter_body(old_hbm, kept_hbm, kidx_hbm, comp3_hbm, counts_hbm, out_hbm,
                  kidx_v, cidx_v, buf0, buf1, cnt_v, sem_l, sem_g, sem_s):
    wid = lax.axis_index("s") * NC + lax.axis_index("c")
    pltpu.sync_copy(kidx_hbm.at[wid], kidx_v)
    pltpu.sync_copy(counts_hbm.at[wid], cnt_v)
    nb = cnt_v[pl.ds(0, 16)][0]

    def koff(b):
        return pl.multiple_of(wid * KPW + b * CBLK, 8)

    def kfire(buf, b):
        return [pltpu.async_copy(buf.at[pl.ds(c * SCAT, SCAT)],
                                 out_hbm.at[kidx_v.at[b * BLK + c]], sem_s)
                for c in range(BLK)]

    def kpair(g, _):
        b0 = 2 * g
        h0 = pltpu.async_copy(kept_hbm.at[pl.ds(koff(b0), CBLK)], buf0, sem_l)
        h1 = pltpu.async_copy(
            kept_hbm.at[pl.ds(koff(b0 + 1), CBLK)], buf1, sem_l)
        h0.wait()
        s0 = kfire(buf0, b0)
        h1.wait()
        s1 = kfire(buf1, b0 + 1)
        for h in s0 + s1:
            h.wait()
        return 0

    lax.fori_loop(0, KNB // 2, kpair, 0)
    hT = pltpu.async_copy(kept_hbm.at[pl.ds(koff(KNB - 1), CBLK)], buf0, sem_l)
    hT.wait()
    for h in kfire(buf0, KNB - 1):
        h.wait()

    def cpair(g, _):
        pltpu.sync_copy(
            comp3_hbm.at[wid, pl.ds(pl.multiple_of(8 * g, 8), 8)], cidx_v)
        g0 = [pltpu.async_copy(old_hbm.at[cidx_v.at[r]],
                               buf0.at[pl.ds(r * SCAT, SCAT)], sem_g)
              for r in range(BLK)]
        g1 = [pltpu.async_copy(old_hbm.at[cidx_v.at[BLK + r]],
                               buf1.at[pl.ds(r * SCAT, SCAT)], sem_g)
              for r in range(BLK)]
        for h in g0:
            h.wait()
        s0 = [pltpu.async_copy(buf0.at[pl.ds(r * SCAT, SCAT)],
                               out_hbm.at[cidx_v.at[r]], sem_s)
              for r in range(BLK)]
        for h in g1:
            h.wait()
        s1 = [pltpu.async_copy(buf1.at[pl.ds(r * SCAT, SCAT)],
                               out_hbm.at[cidx_v.at[BLK + r]], sem_s)
              for r in range(BLK)]
        for h in s0 + s1:
            h.wait()
        return 0

    lax.fori_loop(0, nb // 2, cpair, 0)


_scatter = functools.partial(
    pl.kernel,
    mesh=_mesh,
    out_type=jax.ShapeDtypeStruct((E_OLD, D), jnp.float32),
    scratch_types=[
        pltpu.VMEM((KPW // SCAT, SCAT), jnp.int32),
        pltpu.VMEM((2 * BLK, SCAT), jnp.int32),
        pltpu.VMEM((CBLK, D), jnp.float32),
        pltpu.VMEM((CBLK, D), jnp.float32),
        pltpu.VMEM((16,), jnp.int32),
        pltpu.SemaphoreType.DMA,
        pltpu.SemaphoreType.DMA,
        pltpu.SemaphoreType.DMA,
    ],
)(_scatter_body)


def kernel(old_features, mask_idx, kept_features, edge_index_old):
    starts = jnp.arange(NW, dtype=jnp.int32) * CAND
    bounds = jnp.concatenate([
        jnp.searchsorted(mask_idx, starts).astype(jnp.int32),
        jnp.full((16,), E_KEPT, jnp.int32),
    ])
    kidx3 = mask_idx.reshape(NW, KPW // SCAT, SCAT)
    comp, counts = _build(mask_idx, bounds)
    comp3 = comp.reshape(NW, CCAP // SCAT, SCAT)
    restored = _scatter(old_features, kept_features, kidx3, comp3, counts)
    return restored, edge_index_old


# speedup vs baseline: 1.1456x; 1.1456x over previous
import functools

import jax
import jax.numpy as jnp
from jax import lax
from jax.experimental import pallas as pl
from jax.experimental.pallas import tpu as pltpu
from jax.experimental.pallas import tpu_sc as plsc

E_OLD = 640000
E_KEPT = 320000
D = 128

NC = 2
NS = 16
NW = NC * NS
CPW = E_OLD // NW
CCH = 5000
KPW = E_KEPT // NW
SCAT = 80
NCHUNK = KPW // SCAT
BLK = 5
NBLK = NCHUNK // BLK
KB = BLK * SCAT

_mesh = plsc.VectorSubcoreMesh(core_axis_name="c", subcore_axis_name="s")


COPY_ROWS = 25600


def _copy_body(old_ref, out_ref):
    out_ref[...] = old_ref[...]


def _copy(old_features):
    return pl.pallas_call(
        _copy_body,
        grid=(E_OLD // COPY_ROWS,),
        in_specs=[pl.BlockSpec((COPY_ROWS, D), lambda i: (i, 0))],
        out_specs=pl.BlockSpec((COPY_ROWS, D), lambda i: (i, 0)),
        out_shape=jax.ShapeDtypeStruct((E_OLD, D), jnp.float32),
    )(old_features)


def _scatter_body(mask3_hbm, kept_hbm, out_hbm, idx_v, buf0, buf1,
                  sem_l, sem_s):
    wid = lax.axis_index("s") * NC + lax.axis_index("c")
    pltpu.sync_copy(mask3_hbm.at[wid], idx_v)

    def kept_at(b):
        off = pl.multiple_of(wid * KPW + lax.min(b, NBLK - 1) * KB, 8)
        return kept_hbm.at[pl.ds(off, KB)]

    def fire(buf, b):
        return [pltpu.async_copy(buf.at[pl.ds(c * SCAT, SCAT)],
                                 out_hbm.at[idx_v.at[b * BLK + c]], sem_s)
                for c in range(BLK)]

    pltpu.async_copy(kept_at(0), buf0, sem_l)

    def pair(g, _):
        b0 = 2 * g
        pltpu.make_async_copy(kept_at(b0), buf0, sem_l).wait()
        h1 = pltpu.async_copy(kept_at(b0 + 1), buf1, sem_l)
        s0 = fire(buf0, b0)
        h1.wait()
        for h in s0:
            h.wait()
        pltpu.async_copy(kept_at(b0 + 2), buf0, sem_l)
        s1 = fire(buf1, b0 + 1)
        for h in s1:
            h.wait()
        return 0

    lax.fori_loop(0, NBLK // 2, pair, 0)
    last = NBLK - 1
    pltpu.make_async_copy(kept_at(last), buf0, sem_l).wait()
    for h in fire(buf0, last):
        h.wait()


_scatter = functools.partial(
    pl.kernel,
    mesh=_mesh,
    out_type=(),
    scratch_types=[
        pltpu.VMEM((NCHUNK, SCAT), jnp.int32),
        pltpu.VMEM((KB, D), jnp.float32),
        pltpu.VMEM((KB, D), jnp.float32),
        pltpu.SemaphoreType.DMA,
        pltpu.SemaphoreType.DMA,
    ],
)(_scatter_body)


def kernel(old_features, mask_idx, kept_features, edge_index_old):
    mask3 = mask_idx.reshape(NW, NCHUNK, SCAT)
    copied = _copy(old_features)
    out_ref = jax.new_ref(copied)
    _scatter(mask3, kept_features, out_ref)
    return out_ref[...], edge_index_old
